# Initial kernel scaffold; baseline (speedup 1.0000x reference)
import jax
import jax.numpy as jnp
from jax.experimental import pallas as pl


def kernel(x, edge_index, W, att_src, att_dst, b_conv, Wl, bl):
    raise NotImplementedError("write your pallas kernel here")


# speedup vs baseline: 26.2944x; 26.2944x over previous
import functools

import jax
import jax.numpy as jnp
from jax import lax
from jax.experimental import pallas as pl
from jax.experimental.pallas import tpu as pltpu
from jax.experimental.pallas import tpu_sc as plsc

N = 10000
E = 320000
H = 128

NC = 2
NS = 16
NW = NC * NS
EPW = E // NW
CHUNK = 80
NCH = EPW // CHUNK
NP = 10240
RPT = NP // NS
BLK = 1000
GRID = N // BLK

_NEG_SLOPE = 0.2


def _prologue_body(x_ref, w_ref, as_ref, ad_ref,
                   xw_ref, asrc_ref, adst_ref, mvec_ref, mx_ref):
    i = pl.program_id(0)
    xw = jnp.dot(x_ref[...], w_ref[...], preferred_element_type=jnp.float32)
    xw_ref[...] = xw
    a_s = jnp.dot(xw, as_ref[...], preferred_element_type=jnp.float32)
    a_d = jnp.dot(xw, ad_ref[...], preferred_element_type=jnp.float32)
    asrc_ref[...] = a_s
    adst_ref[...] = a_d

    @pl.when(i == 0)
    def _():
        mx_ref[0] = -jnp.inf
        mx_ref[1] = -jnp.inf

    mx_ref[0] = jnp.maximum(mx_ref[0], jnp.max(a_s))
    mx_ref[1] = jnp.maximum(mx_ref[1], jnp.max(a_d))

    @pl.when(i == GRID - 1)
    def _():
        t = mx_ref[0] + mx_ref[1]
        m = jnp.maximum(t, _NEG_SLOPE * t)
        mvec_ref[...] = jnp.full((1, H), m, jnp.float32)


def _prologue(x, w, att_src, att_dst):
    return pl.pallas_call(
        _prologue_body,
        grid=(GRID,),
        in_specs=[
            pl.BlockSpec((BLK, H), lambda i: (i, 0)),
            pl.BlockSpec((H, H), lambda i: (0, 0)),
            pl.BlockSpec((H, 1), lambda i: (0, 0)),
            pl.BlockSpec((H, 1), lambda i: (0, 0)),
        ],
        out_specs=[
            pl.BlockSpec((BLK, H), lambda i: (i, 0)),
            pl.BlockSpec((BLK, 1), lambda i: (i, 0)),
            pl.BlockSpec((BLK, 1), lambda i: (i, 0)),
            pl.BlockSpec((1, H), lambda i: (0, 0)),
        ],
        out_shape=[
            jax.ShapeDtypeStruct((N, H), jnp.float32),
            jax.ShapeDtypeStruct((N, 1), jnp.float32),
            jax.ShapeDtypeStruct((N, 1), jnp.float32),
            jax.ShapeDtypeStruct((1, H), jnp.float32),
        ],
        scratch_shapes=[pltpu.SMEM((2,), jnp.float32)],
    )(x, w, att_src, att_dst)


def _edge_body(src_hbm, dst_hbm, asrc_hbm, adst_hbm, xw_hbm, m_hbm,
               num_hbm, den_hbm,
               sidx_v, didx_v, as_v, ad_v, m_v, den_v, rows_v, num_sh,
               sem_s, sem_d, sem_r):
    c = lax.axis_index("c")
    s = lax.axis_index("s")
    wid = s * NC + c

    pltpu.sync_copy(m_hbm.at[0, pl.ds(0, 16)], m_v)

    zeros16 = jnp.zeros((16,), jnp.float32)

    def _zden(i):
        den_v[pl.ds(i * 16, 16)] = zeros16
    lax.fori_loop(0, NP // 16, lambda i, _: (_zden(i), 0)[1], 0)

    def _zrow(i):
        for k in range(H // 16):
            rows_v[i, pl.ds(k * 16, 16)] = zeros16
    lax.fori_loop(0, CHUNK, lambda i, _: (_zrow(i), 0)[1], 0)
    for q in range(RPT // CHUNK):
        pltpu.sync_copy(rows_v, num_sh.at[pl.ds(s * RPT + q * CHUNK, CHUNK)])
    plsc.subcore_barrier()

    mvec = m_v[...]

    def _chunk(j, _):
        pltpu.sync_copy(src_hbm.at[wid, j], sidx_v.at[0])
        pltpu.sync_copy(dst_hbm.at[wid, j], didx_v.at[0])
        cp_s = pltpu.async_copy(asrc_hbm.at[sidx_v.at[0]], as_v, sem_s)
        cp_d = pltpu.async_copy(adst_hbm.at[didx_v.at[0]], ad_v, sem_d)
        cp_r = pltpu.async_copy(xw_hbm.at[sidx_v.at[0]], rows_v, sem_r)
        cp_s.wait()
        cp_d.wait()

        exgs = []
        for g in range(CHUNK // 16):
            dg = didx_v[0, pl.ds(g * 16, 16)]
            av = as_v[pl.ds(g * 16, 16)]
            bv = ad_v[pl.ds(g * 16, 16)]
            e = av + bv
            e = jnp.maximum(e, _NEG_SLOPE * e)
            ex = jnp.exp(e - mvec)
            plsc.addupdate_scatter(den_v, [dg], ex)
            exgs.append(ex)

        cp_r.wait()

        for g in range(CHUNK // 16):
            for l in range(16):
                r = g * 16 + l
                sc = exgs[g][l]
                for k in range(H // 16):
                    col = pl.ds(k * 16, 16)
                    rows_v[r, col] = rows_v[r, col] * sc

        pltpu.sync_copy(rows_v, num_sh.at[didx_v.at[0]], add=True)
        return 0

    lax.fori_loop(0, NCH, _chunk, 0)

    plsc.subcore_barrier()
    pltpu.sync_copy(num_sh.at[pl.ds(s * RPT, RPT)],
                    num_hbm.at[c, pl.ds(s * RPT, RPT)])
    pltpu.sync_copy(den_v, den_hbm.at[wid])


def _edge_pass(src3d, dst3d, a_src, a_dst, xw, mvec):
    mesh = plsc.VectorSubcoreMesh(core_axis_name="c", subcore_axis_name="s")
    f = pl.kernel(
        _edge_body,
        out_type=[
            jax.ShapeDtypeStruct((NC, NP, H), jnp.float32),
            jax.ShapeDtypeStruct((NW, NP), jnp.float32),
        ],
        mesh=mesh,
        compiler_params=pltpu.CompilerParams(needs_layout_passes=False),
        scratch_types=[
            pltpu.VMEM((1, CHUNK), jnp.int32),
            pltpu.VMEM((1, CHUNK), jnp.int32),
            pltpu.VMEM((CHUNK,), jnp.float32),
            pltpu.VMEM((CHUNK,), jnp.float32),
            pltpu.VMEM((16,), jnp.float32),
            pltpu.VMEM((NP,), jnp.float32),
            pltpu.VMEM((CHUNK, H), jnp.float32),
            pltpu.VMEM_SHARED((NP, H), jnp.float32),
            pltpu.SemaphoreType.DMA,
            pltpu.SemaphoreType.DMA,
            pltpu.SemaphoreType.DMA,
        ],
    )
    return f(src3d, dst3d, a_src, a_dst, xw, mvec)


def _epilogue_body(xw_ref, as_ref, ad_ref, mv_ref, num_ref, den_ref,
                   bc_ref, wl_ref, bl_ref, y_ref, acc_ref, dcol_ref):
    i = pl.program_id(0)

    @pl.when(i == 0)
    def _():
        dcol_ref[...] = lax.dot_general(
            den_ref[...], jnp.ones((NW, 1), jnp.float32),
            (((0,), (0,)), ((), ())), preferred_element_type=jnp.float32)

    m11 = mv_ref[...][:, :1]
    a = as_ref[...] + ad_ref[...]
    ex_self = jnp.exp(jnp.maximum(a, _NEG_SLOPE * a) - m11)
    xw = xw_ref[...]
    num = num_ref[0] + num_ref[1] + ex_self * xw
    den = dcol_ref[pl.ds(i * BLK, BLK), :] + ex_self + 1e-16
    h = jnp.maximum(num / den + bc_ref[...], 0.0)

    @pl.when(i == 0)
    def _():
        acc_ref[...] = jnp.zeros((1, H), jnp.float32)

    acc_ref[...] += jnp.sum(h, axis=0, keepdims=True)

    @pl.when(i == GRID - 1)
    def _():
        y_ref[...] = jnp.dot(acc_ref[...] * (1.0 / N), wl_ref[...],
                             preferred_element_type=jnp.float32) + bl_ref[...]


def _epilogue(xw, a_src, a_dst, mvec, num_p, den_p, b_conv, wl, bl):
    return pl.pallas_call(
        _epilogue_body,
        grid=(GRID,),
        in_specs=[
            pl.BlockSpec((BLK, H), lambda i: (i, 0)),
            pl.BlockSpec((BLK, 1), lambda i: (i, 0)),
            pl.BlockSpec((BLK, 1), lambda i: (i, 0)),
            pl.BlockSpec((1, H), lambda i: (0, 0)),
            pl.BlockSpec((NC, BLK, H), lambda i: (0, i, 0)),
            pl.BlockSpec((NW, NP), lambda i: (0, 0)),
            pl.BlockSpec((1, H), lambda i: (0, 0)),
            pl.BlockSpec((H, H), lambda i: (0, 0)),
            pl.BlockSpec((1, H), lambda i: (0, 0)),
        ],
        out_specs=pl.BlockSpec((1, H), lambda i: (0, 0)),
        out_shape=jax.ShapeDtypeStruct((1, H), jnp.float32),
        scratch_shapes=[pltpu.VMEM((1, H), jnp.float32),
                        pltpu.VMEM((NP, 1), jnp.float32)],
    )(xw, a_src, a_dst, mvec, num_p, den_p, b_conv, wl, bl)


@jax.jit
def kernel(x, edge_index, W, att_src, att_dst, b_conv, Wl, bl):
    src3d = edge_index[0].reshape(NW, NCH, CHUNK)
    dst3d = edge_index[1].reshape(NW, NCH, CHUNK)

    xw, a_src, a_dst, mvec = _prologue(
        x, W, att_src.reshape(H, 1), att_dst.reshape(H, 1))

    num_p, den_p = _edge_pass(src3d, dst3d,
                              a_src.reshape(N), a_dst.reshape(N), xw, mvec)

    return _epilogue(xw, a_src, a_dst, mvec, num_p, den_p,
                     b_conv.reshape(1, H), Wl, bl.reshape(1, H))


# speedup vs baseline: 31.8031x; 1.2095x over previous
import functools

import jax
import jax.numpy as jnp
from jax import lax
from jax.experimental import pallas as pl
from jax.experimental.pallas import tpu as pltpu
from jax.experimental.pallas import tpu_sc as plsc

N = 10000
E = 320000
H = 128

NC = 2
NS = 16
NW = NC * NS
EPW = E // NW
CHUNK = 80
NCH = EPW // CHUNK
NP = 10240
RPT = NP // NS
BLK = 1000
GRID = N // BLK

_NEG_SLOPE = 0.2


def _prologue_body(x_ref, w_ref, as_ref, ad_ref,
                   xw_ref, asrc_ref, adst_ref, mvec_ref, mx_ref):
    i = pl.program_id(0)
    xw = jnp.dot(x_ref[...], w_ref[...], preferred_element_type=jnp.float32)
    xw_ref[...] = xw
    a_s = jnp.dot(xw, as_ref[...], preferred_element_type=jnp.float32)
    a_d = jnp.dot(xw, ad_ref[...], preferred_element_type=jnp.float32)
    asrc_ref[...] = a_s
    adst_ref[...] = a_d

    @pl.when(i == 0)
    def _():
        mx_ref[0] = -jnp.inf
        mx_ref[1] = -jnp.inf

    mx_ref[0] = jnp.maximum(mx_ref[0], jnp.max(a_s))
    mx_ref[1] = jnp.maximum(mx_ref[1], jnp.max(a_d))

    @pl.when(i == GRID - 1)
    def _():
        t = mx_ref[0] + mx_ref[1]
        m = jnp.maximum(t, _NEG_SLOPE * t)
        mvec_ref[...] = jnp.full((1, H), m, jnp.float32)


def _prologue(x, w, att_src, att_dst):
    return pl.pallas_call(
        _prologue_body,
        grid=(GRID,),
        in_specs=[
            pl.BlockSpec((BLK, H), lambda i: (i, 0)),
            pl.BlockSpec((H, H), lambda i: (0, 0)),
            pl.BlockSpec((H, 1), lambda i: (0, 0)),
            pl.BlockSpec((H, 1), lambda i: (0, 0)),
        ],
        out_specs=[
            pl.BlockSpec((BLK, H), lambda i: (i, 0)),
            pl.BlockSpec((BLK, 1), lambda i: (i, 0)),
            pl.BlockSpec((BLK, 1), lambda i: (i, 0)),
            pl.BlockSpec((1, H), lambda i: (0, 0)),
        ],
        out_shape=[
            jax.ShapeDtypeStruct((N, H), jnp.float32),
            jax.ShapeDtypeStruct((N, 1), jnp.float32),
            jax.ShapeDtypeStruct((N, 1), jnp.float32),
            jax.ShapeDtypeStruct((1, H), jnp.float32),
        ],
        scratch_shapes=[pltpu.SMEM((2,), jnp.float32)],
    )(x, w, att_src, att_dst)


def _edge_body(src_hbm, dst_hbm, asrc_hbm, adst_hbm, xw_hbm, m_hbm,
               num_hbm, den_hbm,
               sidx_v, didx_v, as_v, ad_v, m_v, den_v, rows_v, num_sh,
               sem_s0, sem_s1, sem_d0, sem_d1, sem_r0, sem_r1):
    c = lax.axis_index("c")
    s = lax.axis_index("s")
    wid = s * NC + c
    sem_s = (sem_s0, sem_s1)
    sem_d = (sem_d0, sem_d1)
    sem_r = (sem_r0, sem_r1)

    pltpu.sync_copy(m_hbm.at[0, pl.ds(0, 16)], m_v)

    zeros16 = jnp.zeros((16,), jnp.float32)

    def _zden(i):
        den_v[pl.ds(i * 16, 16)] = zeros16
    lax.fori_loop(0, NP // 16, lambda i, _: (_zden(i), 0)[1], 0)

    def _zrow(i):
        for k in range(H // 16):
            rows_v[0, i, pl.ds(k * 16, 16)] = zeros16
    lax.fori_loop(0, CHUNK, lambda i, _: (_zrow(i), 0)[1], 0)
    for q in range(RPT // CHUNK):
        pltpu.sync_copy(rows_v.at[0],
                        num_sh.at[pl.ds(s * RPT + q * CHUNK, CHUNK)])
    plsc.subcore_barrier()

    mvec = m_v[...]

    def _prefetch(j, b):
        pltpu.sync_copy(src_hbm.at[wid, j], sidx_v.at[b])
        pltpu.sync_copy(dst_hbm.at[wid, j], didx_v.at[b])
        pltpu.async_copy(asrc_hbm.at[sidx_v.at[b]], as_v.at[b], sem_s[b])
        pltpu.async_copy(adst_hbm.at[didx_v.at[b]], ad_v.at[b], sem_d[b])
        pltpu.async_copy(xw_hbm.at[sidx_v.at[b]], rows_v.at[b], sem_r[b])

    def _wait(b):
        pltpu.make_async_copy(
            asrc_hbm.at[sidx_v.at[b]], as_v.at[b], sem_s[b]).wait()
        pltpu.make_async_copy(
            adst_hbm.at[didx_v.at[b]], ad_v.at[b], sem_d[b]).wait()

    def _consume(b):
        exgs = []
        for g in range(CHUNK // 16):
            dg = didx_v[b, pl.ds(g * 16, 16)]
            av = as_v[b, pl.ds(g * 16, 16)]
            bv = ad_v[b, pl.ds(g * 16, 16)]
            e = av + bv
            e = jnp.maximum(e, _NEG_SLOPE * e)
            ex = jnp.exp(e - mvec)
            plsc.addupdate_scatter(den_v, [dg], ex)
            exgs.append(ex)

        pltpu.make_async_copy(
            xw_hbm.at[sidx_v.at[b]], rows_v.at[b], sem_r[b]).wait()

        for g in range(CHUNK // 16):
            for l in range(16):
                r = g * 16 + l
                sc = exgs[g][l]
                for k in range(H // 16):
                    col = pl.ds(k * 16, 16)
                    rows_v[b, r, col] = rows_v[b, r, col] * sc

        pltpu.sync_copy(rows_v.at[b], num_sh.at[didx_v.at[b]], add=True)

    _prefetch(0, 0)

    def _pair(t, _):
        j0 = 2 * t
        _wait(0)
        _prefetch(j0 + 1, 1)
        _consume(0)
        _wait(1)
        _prefetch(j0 + 2, 0)
        _consume(1)
        return 0

    lax.fori_loop(0, (NCH - 1) // 2, _pair, 0)

    _wait(0)
    _consume(0)

    plsc.subcore_barrier()
    pltpu.sync_copy(num_sh.at[pl.ds(s * RPT, RPT)],
                    num_hbm.at[c, pl.ds(s * RPT, RPT)])
    pltpu.sync_copy(den_v, den_hbm.at[wid])


def _edge_pass(src3d, dst3d, a_src, a_dst, xw, mvec):
    mesh = plsc.VectorSubcoreMesh(core_axis_name="c", subcore_axis_name="s")
    f = pl.kernel(
        _edge_body,
        out_type=[
            jax.ShapeDtypeStruct((NC, NP, H), jnp.float32),
            jax.ShapeDtypeStruct((NW, NP), jnp.float32),
        ],
        mesh=mesh,
        compiler_params=pltpu.CompilerParams(needs_layout_passes=False),
        scratch_types=[
            pltpu.VMEM((2, CHUNK), jnp.int32),
            pltpu.VMEM((2, CHUNK), jnp.int32),
            pltpu.VMEM((2, CHUNK), jnp.float32),
            pltpu.VMEM((2, CHUNK), jnp.float32),
            pltpu.VMEM((16,), jnp.float32),
            pltpu.VMEM((NP,), jnp.float32),
            pltpu.VMEM((2, CHUNK, H), jnp.float32),
            pltpu.VMEM_SHARED((NP, H), jnp.float32),
            pltpu.SemaphoreType.DMA,
            pltpu.SemaphoreType.DMA,
            pltpu.SemaphoreType.DMA,
            pltpu.SemaphoreType.DMA,
            pltpu.SemaphoreType.DMA,
            pltpu.SemaphoreType.DMA,
        ],
    )
    return f(src3d, dst3d, a_src, a_dst, xw, mvec)


def _epilogue_body(xw_ref, as_ref, ad_ref, mv_ref, num_ref, den_ref,
                   bc_ref, wl_ref, bl_ref, y_ref, acc_ref, dcol_ref):
    i = pl.program_id(0)

    @pl.when(i == 0)
    def _():
        dcol_ref[...] = lax.dot_general(
            den_ref[...], jnp.ones((NW, 1), jnp.float32),
            (((0,), (0,)), ((), ())), preferred_element_type=jnp.float32)

    m11 = mv_ref[...][:, :1]
    a = as_ref[...] + ad_ref[...]
    ex_self = jnp.exp(jnp.maximum(a, _NEG_SLOPE * a) - m11)
    xw = xw_ref[...]
    num = num_ref[0] + num_ref[1] + ex_self * xw
    den = dcol_ref[pl.ds(i * BLK, BLK), :] + ex_self + 1e-16
    h = jnp.maximum(num / den + bc_ref[...], 0.0)

    @pl.when(i == 0)
    def _():
        acc_ref[...] = jnp.zeros((1, H), jnp.float32)

    acc_ref[...] += jnp.sum(h, axis=0, keepdims=True)

    @pl.when(i == GRID - 1)
    def _():
        y_ref[...] = jnp.dot(acc_ref[...] * (1.0 / N), wl_ref[...],
                             preferred_element_type=jnp.float32) + bl_ref[...]


def _epilogue(xw, a_src, a_dst, mvec, num_p, den_p, b_conv, wl, bl):
    return pl.pallas_call(
        _epilogue_body,
        grid=(GRID,),
        in_specs=[
            pl.BlockSpec((BLK, H), lambda i: (i, 0)),
            pl.BlockSpec((BLK, 1), lambda i: (i, 0)),
            pl.BlockSpec((BLK, 1), lambda i: (i, 0)),
            pl.BlockSpec((1, H), lambda i: (0, 0)),
            pl.BlockSpec((NC, BLK, H), lambda i: (0, i, 0)),
            pl.BlockSpec((NW, NP), lambda i: (0, 0)),
            pl.BlockSpec((1, H), lambda i: (0, 0)),
            pl.BlockSpec((H, H), lambda i: (0, 0)),
            pl.BlockSpec((1, H), lambda i: (0, 0)),
        ],
        out_specs=pl.BlockSpec((1, H), lambda i: (0, 0)),
        out_shape=jax.ShapeDtypeStruct((1, H), jnp.float32),
        scratch_shapes=[pltpu.VMEM((1, H), jnp.float32),
                        pltpu.VMEM((NP, 1), jnp.float32)],
    )(xw, a_src, a_dst, mvec, num_p, den_p, b_conv, wl, bl)


@jax.jit
def kernel(x, edge_index, W, att_src, att_dst, b_conv, Wl, bl):
    src3d = edge_index[0].reshape(NW, NCH, CHUNK)
    dst3d = edge_index[1].reshape(NW, NCH, CHUNK)

    xw, a_src, a_dst, mvec = _prologue(
        x, W, att_src.reshape(H, 1), att_dst.reshape(H, 1))

    num_p, den_p = _edge_pass(src3d, dst3d,
                              a_src.reshape(N), a_dst.reshape(N), xw, mvec)

    return _epilogue(xw, a_src, a_dst, mvec, num_p, den_p,
                     b_conv.reshape(1, H), Wl, bl.reshape(1, H))


# speedup vs baseline: 34.2150x; 1.0758x over previous
import functools

import jax
import jax.numpy as jnp
from jax import lax
from jax.experimental import pallas as pl
from jax.experimental.pallas import tpu as pltpu
from jax.experimental.pallas import tpu_sc as plsc

N = 10000
E = 320000
H = 128

NC = 2
NS = 16
NW = NC * NS
EPW = E // NW
CHUNK = 80
NCH = EPW // CHUNK
NP = 10240
RPT = NP // NS
BLK = 1000
GRID = N // BLK

_NEG_SLOPE = 0.2


def _prologue_body(x_ref, w_ref, as_ref, ad_ref,
                   xw_ref, asrc_ref, adst_ref, mvec_ref, mx_ref):
    i = pl.program_id(0)
    xw = jnp.dot(x_ref[...], w_ref[...], preferred_element_type=jnp.float32)
    xw_ref[...] = xw
    a_s = jnp.dot(xw, as_ref[...], preferred_element_type=jnp.float32)
    a_d = jnp.dot(xw, ad_ref[...], preferred_element_type=jnp.float32)
    asrc_ref[...] = a_s
    adst_ref[...] = a_d

    @pl.when(i == 0)
    def _():
        mx_ref[0] = -jnp.inf
        mx_ref[1] = -jnp.inf

    mx_ref[0] = jnp.maximum(mx_ref[0], jnp.max(a_s))
    mx_ref[1] = jnp.maximum(mx_ref[1], jnp.max(a_d))

    @pl.when(i == GRID - 1)
    def _():
        t = mx_ref[0] + mx_ref[1]
        m = jnp.maximum(t, _NEG_SLOPE * t)
        mvec_ref[...] = jnp.full((1, H), m, jnp.float32)


def _prologue(x, w, att_src, att_dst):
    return pl.pallas_call(
        _prologue_body,
        grid=(GRID,),
        in_specs=[
            pl.BlockSpec((BLK, H), lambda i: (i, 0)),
            pl.BlockSpec((H, H), lambda i: (0, 0)),
            pl.BlockSpec((H, 1), lambda i: (0, 0)),
            pl.BlockSpec((H, 1), lambda i: (0, 0)),
        ],
        out_specs=[
            pl.BlockSpec((BLK, H), lambda i: (i, 0)),
            pl.BlockSpec((BLK, 1), lambda i: (i, 0)),
            pl.BlockSpec((BLK, 1), lambda i: (i, 0)),
            pl.BlockSpec((1, H), lambda i: (0, 0)),
        ],
        out_shape=[
            jax.ShapeDtypeStruct((N, H), jnp.float32),
            jax.ShapeDtypeStruct((N, 1), jnp.float32),
            jax.ShapeDtypeStruct((N, 1), jnp.float32),
            jax.ShapeDtypeStruct((1, H), jnp.float32),
        ],
        scratch_shapes=[pltpu.SMEM((2,), jnp.float32)],
    )(x, w, att_src, att_dst)


def _edge_body(sd_hbm, asrc_hbm, adst_hbm, xw_hbm, m_hbm,
               num_hbm, den_hbm,
               sd_v, as_v, ad_v, m_v, den_v, rows_v, num_sh,
               sem_s0, sem_s1, sem_d0, sem_d1, sem_r0, sem_r1,
               sem_w0, sem_w1):
    c = lax.axis_index("c")
    s = lax.axis_index("s")
    wid = s * NC + c
    sem_s = (sem_s0, sem_s1)
    sem_d = (sem_d0, sem_d1)
    sem_r = (sem_r0, sem_r1)
    sem_w = (sem_w0, sem_w1)

    pltpu.sync_copy(m_hbm.at[0, pl.ds(0, 16)], m_v)

    zeros16 = jnp.zeros((16,), jnp.float32)
    izeros16 = jnp.zeros((16,), jnp.int32)

    def _zden(i):
        den_v[pl.ds(i * 16, 16)] = zeros16
    lax.fori_loop(0, NP // 16, lambda i, _: (_zden(i), 0)[1], 0)

    def _zrow(i):
        for b in range(2):
            for k in range(H // 16):
                rows_v[b, i, pl.ds(k * 16, 16)] = zeros16
    lax.fori_loop(0, CHUNK, lambda i, _: (_zrow(i), 0)[1], 0)
    for g in range(CHUNK // 16):
        sd_v[1, 1, pl.ds(g * 16, 16)] = izeros16
    for q in range(RPT // CHUNK):
        pltpu.sync_copy(rows_v.at[0],
                        num_sh.at[pl.ds(s * RPT + q * CHUNK, CHUNK)])
    plsc.subcore_barrier()

    mvec = m_v[...]

    def _scatter(b):
        return pltpu.async_copy(rows_v.at[b], num_sh.at[sd_v.at[b, 1]],
                                sem_w[b], add=True)

    def _wait_scatter(b):
        pltpu.make_async_copy(rows_v.at[b], num_sh.at[sd_v.at[b, 1]],
                              sem_w[b]).wait()

    def _prefetch(j, b):
        pltpu.sync_copy(sd_hbm.at[wid, j], sd_v.at[b])
        pltpu.async_copy(asrc_hbm.at[sd_v.at[b, 0]], as_v.at[b], sem_s[b])
        pltpu.async_copy(adst_hbm.at[sd_v.at[b, 1]], ad_v.at[b], sem_d[b])
        pltpu.async_copy(xw_hbm.at[sd_v.at[b, 0]], rows_v.at[b], sem_r[b])

    def _wait(b):
        pltpu.make_async_copy(
            asrc_hbm.at[sd_v.at[b, 0]], as_v.at[b], sem_s[b]).wait()
        pltpu.make_async_copy(
            adst_hbm.at[sd_v.at[b, 1]], ad_v.at[b], sem_d[b]).wait()

    def _consume(b):
        exgs = []
        for g in range(CHUNK // 16):
            dg = sd_v[b, 1, pl.ds(g * 16, 16)]
            av = as_v[b, pl.ds(g * 16, 16)]
            bv = ad_v[b, pl.ds(g * 16, 16)]
            e = av + bv
            e = jnp.maximum(e, _NEG_SLOPE * e)
            ex = jnp.exp(e - mvec)
            plsc.addupdate_scatter(den_v, [dg], ex)
            exgs.append(ex)

        pltpu.make_async_copy(
            xw_hbm.at[sd_v.at[b, 0]], rows_v.at[b], sem_r[b]).wait()

        for g in range(CHUNK // 16):
            for l in range(16):
                r = g * 16 + l
                sc = exgs[g][l]
                for k in range(H // 16):
                    col = pl.ds(k * 16, 16)
                    rows_v[b, r, col] = rows_v[b, r, col] * sc

        _scatter(b)

    _prefetch(0, 0)
    _scatter(1)

    def _pair(t, _):
        j0 = 2 * t
        _wait(0)
        _wait_scatter(1)
        _prefetch(j0 + 1, 1)
        _consume(0)
        _wait(1)
        _wait_scatter(0)
        _prefetch(j0 + 2, 0)
        _consume(1)
        return 0

    lax.fori_loop(0, (NCH - 1) // 2, _pair, 0)

    _wait(0)
    _wait_scatter(1)
    _consume(0)
    _wait_scatter(0)

    plsc.subcore_barrier()
    pltpu.sync_copy(num_sh.at[pl.ds(s * RPT, RPT)],
                    num_hbm.at[c, pl.ds(s * RPT, RPT)])
    pltpu.sync_copy(den_v, den_hbm.at[wid])


def _edge_pass(sd4d, a_src, a_dst, xw, mvec):
    mesh = plsc.VectorSubcoreMesh(core_axis_name="c", subcore_axis_name="s")
    f = pl.kernel(
        _edge_body,
        out_type=[
            jax.ShapeDtypeStruct((NC, NP, H), jnp.float32),
            jax.ShapeDtypeStruct((NW, NP), jnp.float32),
        ],
        mesh=mesh,
        compiler_params=pltpu.CompilerParams(needs_layout_passes=False),
        scratch_types=[
            pltpu.VMEM((2, 2, CHUNK), jnp.int32),
            pltpu.VMEM((2, CHUNK), jnp.float32),
            pltpu.VMEM((2, CHUNK), jnp.float32),
            pltpu.VMEM((16,), jnp.float32),
            pltpu.VMEM((NP,), jnp.float32),
            pltpu.VMEM((2, CHUNK, H), jnp.float32),
            pltpu.VMEM_SHARED((NP, H), jnp.float32),
            pltpu.SemaphoreType.DMA,
            pltpu.SemaphoreType.DMA,
            pltpu.SemaphoreType.DMA,
            pltpu.SemaphoreType.DMA,
            pltpu.SemaphoreType.DMA,
            pltpu.SemaphoreType.DMA,
            pltpu.SemaphoreType.DMA,
            pltpu.SemaphoreType.DMA,
        ],
    )
    return f(sd4d, a_src, a_dst, xw, mvec)


def _epilogue_body(xw_ref, as_ref, ad_ref, mv_ref, num_ref, den_ref,
                   bc_ref, wl_ref, bl_ref, y_ref, acc_ref, dcol_ref):
    i = pl.program_id(0)

    @pl.when(i == 0)
    def _():
        dcol_ref[...] = lax.dot_general(
            den_ref[...], jnp.ones((NW, 1), jnp.float32),
            (((0,), (0,)), ((), ())), preferred_element_type=jnp.float32)

    m11 = mv_ref[...][:, :1]
    a = as_ref[...] + ad_ref[...]
    ex_self = jnp.exp(jnp.maximum(a, _NEG_SLOPE * a) - m11)
    xw = xw_ref[...]
    num = num_ref[0] + num_ref[1] + ex_self * xw
    den = dcol_ref[pl.ds(i * BLK, BLK), :] + ex_self + 1e-16
    h = jnp.maximum(num / den + bc_ref[...], 0.0)

    @pl.when(i == 0)
    def _():
        acc_ref[...] = jnp.zeros((1, H), jnp.float32)

    acc_ref[...] += jnp.sum(h, axis=0, keepdims=True)

    @pl.when(i == GRID - 1)
    def _():
        y_ref[...] = jnp.dot(acc_ref[...] * (1.0 / N), wl_ref[...],
                             preferred_element_type=jnp.float32) + bl_ref[...]


def _epilogue(xw, a_src, a_dst, mvec, num_p, den_p, b_conv, wl, bl):
    return pl.pallas_call(
        _epilogue_body,
        grid=(GRID,),
        in_specs=[
            pl.BlockSpec((BLK, H), lambda i: (i, 0)),
            pl.BlockSpec((BLK, 1), lambda i: (i, 0)),
            pl.BlockSpec((BLK, 1), lambda i: (i, 0)),
            pl.BlockSpec((1, H), lambda i: (0, 0)),
            pl.BlockSpec((NC, BLK, H), lambda i: (0, i, 0)),
            pl.BlockSpec((NW, NP), lambda i: (0, 0)),
            pl.BlockSpec((1, H), lambda i: (0, 0)),
            pl.BlockSpec((H, H), lambda i: (0, 0)),
            pl.BlockSpec((1, H), lambda i: (0, 0)),
        ],
        out_specs=pl.BlockSpec((1, H), lambda i: (0, 0)),
        out_shape=jax.ShapeDtypeStruct((1, H), jnp.float32),
        scratch_shapes=[pltpu.VMEM((1, H), jnp.float32),
                        pltpu.VMEM((NP, 1), jnp.float32)],
    )(xw, a_src, a_dst, mvec, num_p, den_p, b_conv, wl, bl)


@jax.jit
def kernel(x, edge_index, W, att_src, att_dst, b_conv, Wl, bl):
    sd4d = jnp.concatenate(
        [edge_index[0].reshape(NW, NCH, 1, CHUNK),
         edge_index[1].reshape(NW, NCH, 1, CHUNK)], axis=2)

    xw, a_src, a_dst, mvec = _prologue(
        x, W, att_src.reshape(H, 1), att_dst.reshape(H, 1))

    num_p, den_p = _edge_pass(sd4d,
                              a_src.reshape(N), a_dst.reshape(N), xw, mvec)

    return _epilogue(xw, a_src, a_dst, mvec, num_p, den_p,
                     b_conv.reshape(1, H), Wl, bl.reshape(1, H))


# speedup vs baseline: 37.2827x; 1.0897x over previous
import functools

import jax
import jax.numpy as jnp
from jax import lax
from jax.experimental import pallas as pl
from jax.experimental.pallas import tpu as pltpu
from jax.experimental.pallas import tpu_sc as plsc

N = 10000
E = 320000
H = 128

NC = 2
NS = 16
NW = NC * NS
EPW = E // NW
CHUNK = 80
NCH = EPW // CHUNK
NP = 10240
RPT = NP // NS
BLK = 1000
GRID = N // BLK

_NEG_SLOPE = 0.2


def _prologue_body(x_ref, w_ref, as_ref, ad_ref,
                   xw_ref, asrc_ref, adst_ref, mvec_ref, mx_ref):
    i = pl.program_id(0)
    xw = jnp.dot(x_ref[...], w_ref[...], preferred_element_type=jnp.float32)
    xw_ref[...] = xw
    a_s = jnp.dot(xw, as_ref[...], preferred_element_type=jnp.float32)
    a_d = jnp.dot(xw, ad_ref[...], preferred_element_type=jnp.float32)
    asrc_ref[...] = a_s
    adst_ref[...] = a_d

    @pl.when(i == 0)
    def _():
        mx_ref[0] = -jnp.inf
        mx_ref[1] = -jnp.inf

    mx_ref[0] = jnp.maximum(mx_ref[0], jnp.max(a_s))
    mx_ref[1] = jnp.maximum(mx_ref[1], jnp.max(a_d))

    @pl.when(i == GRID - 1)
    def _():
        t = mx_ref[0] + mx_ref[1]
        m = jnp.maximum(t, _NEG_SLOPE * t)
        mvec_ref[...] = jnp.full((1, H), m, jnp.float32)


def _prologue(x, w, att_src, att_dst):
    return pl.pallas_call(
        _prologue_body,
        grid=(GRID,),
        in_specs=[
            pl.BlockSpec((BLK, H), lambda i: (i, 0)),
            pl.BlockSpec((H, H), lambda i: (0, 0)),
            pl.BlockSpec((H, 1), lambda i: (0, 0)),
            pl.BlockSpec((H, 1), lambda i: (0, 0)),
        ],
        out_specs=[
            pl.BlockSpec((BLK, H), lambda i: (i, 0)),
            pl.BlockSpec((BLK, 1), lambda i: (i, 0)),
            pl.BlockSpec((BLK, 1), lambda i: (i, 0)),
            pl.BlockSpec((1, H), lambda i: (0, 0)),
        ],
        out_shape=[
            jax.ShapeDtypeStruct((N, H), jnp.float32),
            jax.ShapeDtypeStruct((N, 1), jnp.float32),
            jax.ShapeDtypeStruct((N, 1), jnp.float32),
            jax.ShapeDtypeStruct((1, H), jnp.float32),
        ],
        scratch_shapes=[pltpu.SMEM((2,), jnp.float32)],
    )(x, w, att_src, att_dst)


def _edge_body(sd_hbm, asrc_hbm, adst_hbm, xw_hbm, m_hbm,
               num_hbm, den_hbm,
               sd_v, as_v, ad_v, m_v, den_v, rows_v, num_sh,
               sem_i0, sem_i1, sem_i2, sem_i3,
               sem_s0, sem_s1, sem_d0, sem_d1, sem_r0, sem_r1,
               sem_w0, sem_w1):
    c = lax.axis_index("c")
    s = lax.axis_index("s")
    wid = s * NC + c
    sem_i = (sem_i0, sem_i1, sem_i2, sem_i3)
    sem_s = (sem_s0, sem_s1)
    sem_d = (sem_d0, sem_d1)
    sem_r = (sem_r0, sem_r1)
    sem_w = (sem_w0, sem_w1)

    pltpu.sync_copy(m_hbm.at[0, pl.ds(0, 16)], m_v)

    zeros16 = jnp.zeros((16,), jnp.float32)
    izeros16 = jnp.zeros((16,), jnp.int32)

    def _zden(i):
        den_v[pl.ds(i * 16, 16)] = zeros16
    lax.fori_loop(0, NP // 16, lambda i, _: (_zden(i), 0)[1], 0)

    def _zrow(i):
        for b in range(2):
            for k in range(H // 16):
                rows_v[b, i, pl.ds(k * 16, 16)] = zeros16
    lax.fori_loop(0, CHUNK, lambda i, _: (_zrow(i), 0)[1], 0)
    for g in range(CHUNK // 16):
        sd_v[3, 1, pl.ds(g * 16, 16)] = izeros16
    for q in range(RPT // CHUNK):
        pltpu.sync_copy(rows_v.at[0],
                        num_sh.at[pl.ds(s * RPT + q * CHUNK, CHUNK)])
    plsc.subcore_barrier()

    mvec = m_v[...]


    def _stage_idx(j, u):
        pltpu.async_copy(sd_hbm.at[wid, j], sd_v.at[u], sem_i[u])

    def _wait_idx(u):
        pltpu.make_async_copy(sd_hbm.at[wid, 0], sd_v.at[u],
                              sem_i[u]).wait()

    def _fire_gathers(u, b):
        pltpu.async_copy(asrc_hbm.at[sd_v.at[u, 0]], as_v.at[b], sem_s[b])
        pltpu.async_copy(adst_hbm.at[sd_v.at[u, 1]], ad_v.at[b], sem_d[b])
        pltpu.async_copy(xw_hbm.at[sd_v.at[u, 0]], rows_v.at[b], sem_r[b])

    def _wait_ga(u, b):
        pltpu.make_async_copy(
            asrc_hbm.at[sd_v.at[u, 0]], as_v.at[b], sem_s[b]).wait()
        pltpu.make_async_copy(
            adst_hbm.at[sd_v.at[u, 1]], ad_v.at[b], sem_d[b]).wait()

    def _scatter(u, b):
        pltpu.async_copy(rows_v.at[b], num_sh.at[sd_v.at[u, 1]],
                         sem_w[b], add=True)

    def _wait_scatter(u, b):
        pltpu.make_async_copy(rows_v.at[b], num_sh.at[sd_v.at[u, 1]],
                              sem_w[b]).wait()

    def _consume(u, b):
        exgs = []
        for g in range(CHUNK // 16):
            dg = sd_v[u, 1, pl.ds(g * 16, 16)]
            av = as_v[b, pl.ds(g * 16, 16)]
            bv = ad_v[b, pl.ds(g * 16, 16)]
            e = av + bv
            e = jnp.maximum(e, _NEG_SLOPE * e)
            ex = jnp.exp(e - mvec)
            plsc.addupdate_scatter(den_v, [dg], ex)
            exgs.append(ex)

        pltpu.make_async_copy(
            xw_hbm.at[sd_v.at[u, 0]], rows_v.at[b], sem_r[b]).wait()

        for g in range(CHUNK // 16):
            for l in range(16):
                r = g * 16 + l
                sc = exgs[g][l]
                for k in range(H // 16):
                    col = pl.ds(k * 16, 16)
                    rows_v[b, r, col] = rows_v[b, r, col] * sc

        _scatter(u, b)

    def _sub_iter(j, u, has_next, has_next2):
        b = u % 2
        _wait_ga(u, b)
        _wait_scatter((u - 1) % 4, 1 - b)
        if has_next:
            _wait_idx((u + 1) % 4)
            _fire_gathers((u + 1) % 4, 1 - b)
        if has_next2:
            _stage_idx(j + 2, (u + 2) % 4)
        _consume(u, b)

    pltpu.sync_copy(sd_hbm.at[wid, 0], sd_v.at[0])
    _stage_idx(1, 1)
    _fire_gathers(0, 0)
    _scatter(3, 1)

    TAIL = 5
    NQ = (NCH - TAIL) // 4

    def _quad(t, _):
        j0 = 4 * t
        for u in range(4):
            _sub_iter(j0 + u, u, True, True)
        return 0

    lax.fori_loop(0, NQ, _quad, 0)

    for jt in range(NCH - TAIL, NCH):
        _sub_iter(jt, jt % 4, jt + 1 < NCH, jt + 2 < NCH)
    _wait_scatter((NCH - 1) % 4, (NCH - 1) % 2)

    plsc.subcore_barrier()
    pltpu.sync_copy(num_sh.at[pl.ds(s * RPT, RPT)],
                    num_hbm.at[c, pl.ds(s * RPT, RPT)])
    pltpu.sync_copy(den_v, den_hbm.at[wid])


def _edge_pass(sd4d, a_src, a_dst, xw, mvec):
    mesh = plsc.VectorSubcoreMesh(core_axis_name="c", subcore_axis_name="s")
    f = pl.kernel(
        _edge_body,
        out_type=[
            jax.ShapeDtypeStruct((NC, NP, H), jnp.float32),
            jax.ShapeDtypeStruct((NW, NP), jnp.float32),
        ],
        mesh=mesh,
        compiler_params=pltpu.CompilerParams(needs_layout_passes=False),
        scratch_types=[
            pltpu.VMEM((4, 2, CHUNK), jnp.int32),
            pltpu.VMEM((2, CHUNK), jnp.float32),
            pltpu.VMEM((2, CHUNK), jnp.float32),
            pltpu.VMEM((16,), jnp.float32),
            pltpu.VMEM((NP,), jnp.float32),
            pltpu.VMEM((2, CHUNK, H), jnp.float32),
            pltpu.VMEM_SHARED((NP, H), jnp.float32),
        ] + [pltpu.SemaphoreType.DMA] * 12,
    )
    return f(sd4d, a_src, a_dst, xw, mvec)


def _epilogue_body(xw_ref, as_ref, ad_ref, mv_ref, num_ref, den_ref,
                   bc_ref, wl_ref, bl_ref, y_ref, acc_ref, dcol_ref):
    i = pl.program_id(0)

    @pl.when(i == 0)
    def _():
        dcol_ref[...] = lax.dot_general(
            den_ref[...], jnp.ones((NW, 1), jnp.float32),
            (((0,), (0,)), ((), ())), preferred_element_type=jnp.float32)

    m11 = mv_ref[...][:, :1]
    a = as_ref[...] + ad_ref[...]
    ex_self = jnp.exp(jnp.maximum(a, _NEG_SLOPE * a) - m11)
    xw = xw_ref[...]
    num = num_ref[0] + num_ref[1] + ex_self * xw
    den = dcol_ref[pl.ds(i * BLK, BLK), :] + ex_self + 1e-16
    h = jnp.maximum(num / den + bc_ref[...], 0.0)

    @pl.when(i == 0)
    def _():
        acc_ref[...] = jnp.zeros((1, H), jnp.float32)

    acc_ref[...] += jnp.sum(h, axis=0, keepdims=True)

    @pl.when(i == GRID - 1)
    def _():
        y_ref[...] = jnp.dot(acc_ref[...] * (1.0 / N), wl_ref[...],
                             preferred_element_type=jnp.float32) + bl_ref[...]


def _epilogue(xw, a_src, a_dst, mvec, num_p, den_p, b_conv, wl, bl):
    return pl.pallas_call(
        _epilogue_body,
        grid=(GRID,),
        in_specs=[
            pl.BlockSpec((BLK, H), lambda i: (i, 0)),
            pl.BlockSpec((BLK, 1), lambda i: (i, 0)),
            pl.BlockSpec((BLK, 1), lambda i: (i, 0)),
            pl.BlockSpec((1, H), lambda i: (0, 0)),
            pl.BlockSpec((NC, BLK, H), lambda i: (0, i, 0)),
            pl.BlockSpec((NW, NP), lambda i: (0, 0)),
            pl.BlockSpec((1, H), lambda i: (0, 0)),
            pl.BlockSpec((H, H), lambda i: (0, 0)),
            pl.BlockSpec((1, H), lambda i: (0, 0)),
        ],
        out_specs=pl.BlockSpec((1, H), lambda i: (0, 0)),
        out_shape=jax.ShapeDtypeStruct((1, H), jnp.float32),
        scratch_shapes=[pltpu.VMEM((1, H), jnp.float32),
                        pltpu.VMEM((NP, 1), jnp.float32)],
    )(xw, a_src, a_dst, mvec, num_p, den_p, b_conv, wl, bl)


@jax.jit
def kernel(x, edge_index, W, att_src, att_dst, b_conv, Wl, bl):
    sd4d = jnp.concatenate(
        [edge_index[0].reshape(NW, NCH, 1, CHUNK),
         edge_index[1].reshape(NW, NCH, 1, CHUNK)], axis=2)

    xw, a_src, a_dst, mvec = _prologue(
        x, W, att_src.reshape(H, 1), att_dst.reshape(H, 1))

    num_p, den_p = _edge_pass(sd4d,
                              a_src.reshape(N), a_dst.reshape(N), xw, mvec)

    return _epilogue(xw, a_src, a_dst, mvec, num_p, den_p,
                     b_conv.reshape(1, H), Wl, bl.reshape(1, H))


# speedup vs baseline: 38.3957x; 1.0299x over previous
import functools

import jax
import jax.numpy as jnp
from jax import lax
from jax.experimental import pallas as pl
from jax.experimental.pallas import tpu as pltpu
from jax.experimental.pallas import tpu_sc as plsc

N = 10000
E = 320000
H = 128

NC = 2
NS = 16
NW = NC * NS
EPW = E // NW
CHUNK = 80
NCH = EPW // CHUNK
NP = 10240
RPT = NP // NS
BLK = 1000
GRID = N // BLK

_NEG_SLOPE = 0.2


def _prologue_body(x_ref, w_ref, as_ref, ad_ref,
                   xw_ref, asrc_ref, adst_ref, mvec_ref, mx_ref):
    i = pl.program_id(0)
    xw = jnp.dot(x_ref[...], w_ref[...], preferred_element_type=jnp.float32)
    xw_ref[...] = xw
    a_s = jnp.dot(xw, as_ref[...], preferred_element_type=jnp.float32)
    a_d = jnp.dot(xw, ad_ref[...], preferred_element_type=jnp.float32)
    asrc_ref[...] = a_s
    adst_ref[...] = a_d

    @pl.when(i == 0)
    def _():
        mx_ref[0] = -jnp.inf
        mx_ref[1] = -jnp.inf

    mx_ref[0] = jnp.maximum(mx_ref[0], jnp.max(a_s))
    mx_ref[1] = jnp.maximum(mx_ref[1], jnp.max(a_d))

    @pl.when(i == GRID - 1)
    def _():
        t = mx_ref[0] + mx_ref[1]
        m = jnp.maximum(t, _NEG_SLOPE * t)
        mvec_ref[...] = jnp.full((1, H), m, jnp.float32)


def _prologue(x, w, att_src, att_dst):
    return pl.pallas_call(
        _prologue_body,
        grid=(GRID,),
        in_specs=[
            pl.BlockSpec((BLK, H), lambda i: (i, 0)),
            pl.BlockSpec((H, H), lambda i: (0, 0)),
            pl.BlockSpec((H, 1), lambda i: (0, 0)),
            pl.BlockSpec((H, 1), lambda i: (0, 0)),
        ],
        out_specs=[
            pl.BlockSpec((BLK, H), lambda i: (i, 0)),
            pl.BlockSpec((BLK, 1), lambda i: (i, 0)),
            pl.BlockSpec((BLK, 1), lambda i: (i, 0)),
            pl.BlockSpec((1, H), lambda i: (0, 0)),
        ],
        out_shape=[
            jax.ShapeDtypeStruct((N, H), jnp.float32),
            jax.ShapeDtypeStruct((N, 1), jnp.float32),
            jax.ShapeDtypeStruct((N, 1), jnp.float32),
            jax.ShapeDtypeStruct((1, H), jnp.float32),
        ],
        scratch_shapes=[pltpu.SMEM((2,), jnp.float32)],
    )(x, w, att_src, att_dst)


def _edge_body(sd_hbm, d5_hbm, asrc_hbm, adst_hbm, xw_hbm, m_hbm,
               num_hbm, den_hbm,
               sd_v, dg5_v, as_v, ad_v, m_v, den_v, rows_v, num_sh,
               sem_i0, sem_i1, sem_i2, sem_i3,
               sem_s0, sem_s1, sem_d0, sem_d1, sem_r0, sem_r1,
               sem_w0, sem_w1):
    c = lax.axis_index("c")
    s = lax.axis_index("s")
    wid = s * NC + c
    sem_i = (sem_i0, sem_i1, sem_i2, sem_i3)
    sem_s = (sem_s0, sem_s1)
    sem_d = (sem_d0, sem_d1)
    sem_r = (sem_r0, sem_r1)
    sem_w = (sem_w0, sem_w1)

    pltpu.sync_copy(m_hbm.at[0, pl.ds(0, 16)], m_v)

    zeros16 = jnp.zeros((16,), jnp.float32)
    izeros16 = jnp.zeros((16,), jnp.int32)

    def _zden(i):
        den_v[pl.ds(i * 16, 16)] = zeros16
    lax.fori_loop(0, NP // 16, lambda i, _: (_zden(i), 0)[1], 0)

    def _zrow(i):
        for b in range(2):
            for k in range(H // 16):
                rows_v[b, i, pl.ds(k * 16, 16)] = zeros16
    lax.fori_loop(0, CHUNK, lambda i, _: (_zrow(i), 0)[1], 0)
    for g in range(CHUNK // 16):
        sd_v[3, 1, pl.ds(g * 16, 16)] = izeros16
    for q in range(RPT // CHUNK):
        pltpu.sync_copy(rows_v.at[0],
                        num_sh.at[pl.ds(s * RPT + q * CHUNK, CHUNK)])
    plsc.subcore_barrier()

    mvec = m_v[...]


    def _stage_idx(j, u):
        pltpu.async_copy(sd_hbm.at[wid, j], sd_v.at[u], sem_i[u])
        pltpu.async_copy(d5_hbm.at[wid, j], dg5_v.at[u], sem_i[u])

    def _wait_idx(u):
        pltpu.make_async_copy(sd_hbm.at[wid, 0], sd_v.at[u],
                              sem_i[u]).wait()
        pltpu.make_async_copy(d5_hbm.at[wid, 0], dg5_v.at[u],
                              sem_i[u]).wait()

    def _fire_gathers(u, b):
        pltpu.async_copy(asrc_hbm.at[sd_v.at[u, 0]], as_v.at[b], sem_s[b])
        pltpu.async_copy(adst_hbm.at[sd_v.at[u, 1]], ad_v.at[b], sem_d[b])
        pltpu.async_copy(xw_hbm.at[sd_v.at[u, 0]], rows_v.at[b], sem_r[b])

    def _wait_ga(u, b):
        pltpu.make_async_copy(
            asrc_hbm.at[sd_v.at[u, 0]], as_v.at[b], sem_s[b]).wait()
        pltpu.make_async_copy(
            adst_hbm.at[sd_v.at[u, 1]], ad_v.at[b], sem_d[b]).wait()

    def _scatter(u, b):
        pltpu.async_copy(rows_v.at[b], num_sh.at[sd_v.at[u, 1]],
                         sem_w[b], add=True)

    def _wait_scatter(u, b):
        pltpu.make_async_copy(rows_v.at[b], num_sh.at[sd_v.at[u, 1]],
                              sem_w[b]).wait()

    def _consume(u, b):
        exgs = []
        for g in range(CHUNK // 16):
            dg = sd_v[u, 1, pl.ds(g * 16, 16)]
            av = as_v[b, pl.ds(g * 16, 16)]
            bv = ad_v[b, pl.ds(g * 16, 16)]
            e = av + bv
            e = jnp.maximum(e, _NEG_SLOPE * e)
            ex = jnp.exp(e - mvec)
            plsc.addupdate_scatter(den_v, [dg], ex)
            exgs.append(ex)

        pltpu.make_async_copy(
            xw_hbm.at[sd_v.at[u, 0]], rows_v.at[b], sem_r[b]).wait()

        for g in range(CHUNK // 16):
            for l in range(16):
                r = g * 16 + l
                sc = exgs[g][l]
                for k in range(H // 16):
                    col = pl.ds(k * 16, 16)
                    rows_v[b, r, col] = rows_v[b, r, col] * sc
            pltpu.async_copy(rows_v.at[b, pl.ds(g * 16, 16)],
                             num_sh.at[dg5_v.at[u, g]],
                             sem_w[b], add=True)

    def _sub_iter(j, u, has_next, has_next2):
        b = u % 2
        _wait_ga(u, b)
        _wait_scatter((u - 1) % 4, 1 - b)
        if has_next:
            _wait_idx((u + 1) % 4)
            _fire_gathers((u + 1) % 4, 1 - b)
        if has_next2:
            _stage_idx(j + 2, (u + 2) % 4)
        _consume(u, b)

    pltpu.sync_copy(sd_hbm.at[wid, 0], sd_v.at[0])
    pltpu.sync_copy(d5_hbm.at[wid, 0], dg5_v.at[0])
    _stage_idx(1, 1)
    _fire_gathers(0, 0)
    _scatter(3, 1)

    TAIL = 5
    NQ = (NCH - TAIL) // 4

    def _quad(t, _):
        j0 = 4 * t
        for u in range(4):
            _sub_iter(j0 + u, u, True, True)
        return 0

    lax.fori_loop(0, NQ, _quad, 0)

    for jt in range(NCH - TAIL, NCH):
        _sub_iter(jt, jt % 4, jt + 1 < NCH, jt + 2 < NCH)
    _wait_scatter((NCH - 1) % 4, (NCH - 1) % 2)

    plsc.subcore_barrier()
    pltpu.sync_copy(num_sh.at[pl.ds(s * RPT, RPT)],
                    num_hbm.at[c, pl.ds(s * RPT, RPT)])
    pltpu.sync_copy(den_v, den_hbm.at[wid])


def _edge_pass(sd4d, d5, a_src, a_dst, xw, mvec):
    mesh = plsc.VectorSubcoreMesh(core_axis_name="c", subcore_axis_name="s")
    f = pl.kernel(
        _edge_body,
        out_type=[
            jax.ShapeDtypeStruct((NC, NP, H), jnp.float32),
            jax.ShapeDtypeStruct((NW, NP), jnp.float32),
        ],
        mesh=mesh,
        compiler_params=pltpu.CompilerParams(needs_layout_passes=False),
        scratch_types=[
            pltpu.VMEM((4, 2, CHUNK), jnp.int32),
            pltpu.VMEM((4, CHUNK // 16, 16), jnp.int32),
            pltpu.VMEM((2, CHUNK), jnp.float32),
            pltpu.VMEM((2, CHUNK), jnp.float32),
            pltpu.VMEM((16,), jnp.float32),
            pltpu.VMEM((NP,), jnp.float32),
            pltpu.VMEM((2, CHUNK, H), jnp.float32),
            pltpu.VMEM_SHARED((NP, H), jnp.float32),
        ] + [pltpu.SemaphoreType.DMA] * 12,
    )
    return f(sd4d, d5, a_src, a_dst, xw, mvec)


def _epilogue_body(xw_ref, as_ref, ad_ref, mv_ref, num_ref, den_ref,
                   bc_ref, wl_ref, bl_ref, y_ref, acc_ref, dcol_ref):
    i = pl.program_id(0)

    @pl.when(i == 0)
    def _():
        dcol_ref[...] = lax.dot_general(
            den_ref[...], jnp.ones((NW, 1), jnp.float32),
            (((0,), (0,)), ((), ())), preferred_element_type=jnp.float32)

    m11 = mv_ref[...][:, :1]
    a = as_ref[...] + ad_ref[...]
    ex_self = jnp.exp(jnp.maximum(a, _NEG_SLOPE * a) - m11)
    xw = xw_ref[...]
    num = num_ref[0] + num_ref[1] + ex_self * xw
    den = dcol_ref[pl.ds(i * BLK, BLK), :] + ex_self + 1e-16
    h = jnp.maximum(num / den + bc_ref[...], 0.0)

    @pl.when(i == 0)
    def _():
        acc_ref[...] = jnp.zeros((1, H), jnp.float32)

    acc_ref[...] += jnp.sum(h, axis=0, keepdims=True)

    @pl.when(i == GRID - 1)
    def _():
        y_ref[...] = jnp.dot(acc_ref[...] * (1.0 / N), wl_ref[...],
                             preferred_element_type=jnp.float32) + bl_ref[...]


def _epilogue(xw, a_src, a_dst, mvec, num_p, den_p, b_conv, wl, bl):
    return pl.pallas_call(
        _epilogue_body,
        grid=(GRID,),
        in_specs=[
            pl.BlockSpec((BLK, H), lambda i: (i, 0)),
            pl.BlockSpec((BLK, 1), lambda i: (i, 0)),
            pl.BlockSpec((BLK, 1), lambda i: (i, 0)),
            pl.BlockSpec((1, H), lambda i: (0, 0)),
            pl.BlockSpec((NC, BLK, H), lambda i: (0, i, 0)),
            pl.BlockSpec((NW, NP), lambda i: (0, 0)),
            pl.BlockSpec((1, H), lambda i: (0, 0)),
            pl.BlockSpec((H, H), lambda i: (0, 0)),
            pl.BlockSpec((1, H), lambda i: (0, 0)),
        ],
        out_specs=pl.BlockSpec((1, H), lambda i: (0, 0)),
        out_shape=jax.ShapeDtypeStruct((1, H), jnp.float32),
        scratch_shapes=[pltpu.VMEM((1, H), jnp.float32),
                        pltpu.VMEM((NP, 1), jnp.float32)],
    )(xw, a_src, a_dst, mvec, num_p, den_p, b_conv, wl, bl)


@jax.jit
def kernel(x, edge_index, W, att_src, att_dst, b_conv, Wl, bl):
    sd4d = jnp.concatenate(
        [edge_index[0].reshape(NW, NCH, 1, CHUNK),
         edge_index[1].reshape(NW, NCH, 1, CHUNK)], axis=2)
    d5 = edge_index[1].reshape(NW, NCH, CHUNK // 16, 16)

    xw, a_src, a_dst, mvec = _prologue(
        x, W, att_src.reshape(H, 1), att_dst.reshape(H, 1))

    num_p, den_p = _edge_pass(sd4d, d5,
                              a_src.reshape(N), a_dst.reshape(N), xw, mvec)

    return _epilogue(xw, a_src, a_dst, mvec, num_p, den_p,
                     b_conv.reshape(1, H), Wl, bl.reshape(1, H))
